# Initial kernel scaffold; baseline (speedup 1.0000x reference)
import jax
import jax.numpy as jnp
from jax.experimental import pallas as pl


def kernel(z, e_feat, adj, enc, W_msg_0, b_msg_0, W_upd_0, b_upd_0, W_msg_1, b_msg_1, W_upd_1, b_upd_1, coef):
    raise NotImplementedError("write your pallas kernel here")


# speedup vs baseline: 1.2651x; 1.2651x over previous
import functools

import jax
import jax.numpy as jnp
from jax import lax
from jax.experimental import pallas as pl
from jax.experimental.pallas import tpu as pltpu
from jax.experimental.pallas import tpu_sc as plsc

N = 10000
E = 320000
D = 128
ED = 16
LD = 128

NC = 2
NS = 16
NW = NC * NS
EPAD = 327680
ET = EPAD // NW
K = 128
NCHUNK = ET // K
NPAD = 10240
NROW = NPAD // NS
ZR = 128

_NEG = -1e30


def _node_tables_body(z_ref, w0s_ref, w0d_ref, w1s_ref, w1d_ref,
                      a0_ref, b0_ref, a1_ref, b1_ref):
    zb = z_ref[...]
    dot = lambda w: lax.dot_general(zb, w, (((1,), (0,)), ((), ())),
                                    precision=lax.Precision.HIGHEST,
                                    preferred_element_type=jnp.float32)
    a0_ref[...] = dot(w0s_ref[...])
    b0_ref[...] = dot(w0d_ref[...])
    a1_ref[...] = dot(w1s_ref[...])
    b1_ref[...] = dot(w1d_ref[...])


def _node_tables(z, w0s, w0d, w1s, w1d):
    bn = 1000
    grid = N // bn
    wspec = pl.BlockSpec((D, LD), lambda i: (0, 0))
    ospec = pl.BlockSpec((bn, LD), lambda i: (i, 0))
    return pl.pallas_call(
        _node_tables_body,
        grid=(grid,),
        in_specs=[pl.BlockSpec((bn, D), lambda i: (i, 0)), wspec, wspec, wspec, wspec],
        out_specs=[ospec, ospec, ospec, ospec],
        out_shape=[jax.ShapeDtypeStruct((N, LD), jnp.float32)] * 4,
    )(z, w0s, w0d, w1s, w1d)


def _edge_tables_body(ef_ref, w0e_ref, b0_ref, w1e_ref, b1_ref, c0_ref, c1_ref):
    i = pl.program_id(0)
    eb = ef_ref[...]
    rows = i * eb.shape[0] + lax.broadcasted_iota(jnp.int32, (eb.shape[0], 1), 0)
    valid = rows < E
    dot = lambda w: lax.dot_general(eb, w, (((1,), (0,)), ((), ())),
                                    precision=lax.Precision.HIGHEST,
                                    preferred_element_type=jnp.float32)
    c0 = dot(w0e_ref[...]) + b0_ref[...]
    c1 = dot(w1e_ref[...]) + b1_ref[...]
    c0_ref[...] = jnp.where(valid, c0, _NEG)
    c1_ref[...] = jnp.where(valid, c1, _NEG)


def _edge_tables(e_feat_pad, w0e, bm0, w1e, bm1):
    be = 2048
    grid = EPAD // be
    wspec = pl.BlockSpec((ED, LD), lambda i: (0, 0))
    bspec = pl.BlockSpec((1, LD), lambda i: (0, 0))
    ospec = pl.BlockSpec((be, LD), lambda i: (i, 0))
    return pl.pallas_call(
        _edge_tables_body,
        grid=(grid,),
        in_specs=[pl.BlockSpec((be, ED), lambda i: (i, 0)), wspec, bspec, wspec, bspec],
        out_specs=[ospec, ospec],
        out_shape=[jax.ShapeDtypeStruct((EPAD, LD), jnp.float32)] * 2,
    )(e_feat_pad, w0e, bm0, w1e, bm1)


def _edge_pass_body(src_hbm, dst_hbm, a0, b0, c0, a1, b1, c1, out_hbm,
                    sidx, didx, msg, zbuf, agg, sem_a, sem_b):
    c = lax.axis_index("c")
    s = lax.axis_index("s")
    base = (c * NS + s) * ET

    z16 = jnp.zeros((16,), jnp.float32)

    @pl.loop(0, ZR)
    def _zr(r):
        for j in range(LD // 16):
            zbuf[r, pl.ds(j * 16, 16)] = z16

    for p, (ta, tb, tc) in enumerate(((a0, b0, c0), (a1, b1, c1))):
        for r in range(NROW // ZR):
            pltpu.sync_copy(zbuf, agg.at[pl.ds(s * NROW + r * ZR, ZR)])
        plsc.subcore_barrier()

        @pl.loop(0, NCHUNK)
        def _chunk(j):
            e0 = base + j * K
            pltpu.sync_copy(src_hbm.at[pl.ds(e0, K)], sidx)
            pltpu.sync_copy(dst_hbm.at[pl.ds(e0, K)], didx)
            pltpu.sync_copy(tc.at[pl.ds(e0, K)], msg)
            pltpu.async_copy(ta.at[sidx], msg, sem_a, add=True).wait()
            pltpu.async_copy(tb.at[didx], msg, sem_b, add=True).wait()

            @pl.loop(0, K)
            def _relu(r):
                for j2 in range(LD // 16):
                    v = msg[r, pl.ds(j2 * 16, 16)]
                    msg[r, pl.ds(j2 * 16, 16)] = jnp.maximum(v, 0.0)

            pltpu.sync_copy(msg, agg.at[didx], add=True)

        plsc.subcore_barrier()
        pltpu.sync_copy(agg.at[pl.ds(s * NROW, NROW)],
                        out_hbm.at[c, p, pl.ds(s * NROW, NROW)])
        plsc.subcore_barrier()


def _edge_pass(src, dst, a0, b0, c0, a1, b1, c1):
    mesh = plsc.VectorSubcoreMesh(core_axis_name="c", subcore_axis_name="s",
                                  num_cores=NC, num_subcores=NS)
    kern = pl.kernel(
        _edge_pass_body,
        out_type=jax.ShapeDtypeStruct((NC, 2, NPAD, LD), jnp.float32),
        mesh=mesh,
        scratch_types=[
            pltpu.VMEM((K,), jnp.int32),
            pltpu.VMEM((K,), jnp.int32),
            pltpu.VMEM((K, LD), jnp.float32),
            pltpu.VMEM((ZR, LD), jnp.float32),
            pltpu.VMEM_SHARED((NPAD, LD), jnp.float32),
            pltpu.SemaphoreType.DMA,
            pltpu.SemaphoreType.DMA,
        ],
    )
    return kern(src, dst, a0, b0, c0, a1, b1, c1)


def _update_body(enc_ref, part_ref, w0e_ref, w0a_ref, bu0_ref,
                 w1e_ref, w1a_ref, bu1_ref, coef_ref, out_ref):
    encb = enc_ref[...]
    agg0 = part_ref[0, 0] + part_ref[1, 0]
    agg1 = part_ref[0, 1] + part_ref[1, 1]
    dot = lambda x, w: lax.dot_general(x, w, (((1,), (0,)), ((), ())),
                                       precision=lax.Precision.HIGHEST,
                                       preferred_element_type=jnp.float32)
    u0 = jnp.maximum(dot(encb, w0e_ref[...]) + dot(agg0, w0a_ref[...]) + bu0_ref[...], 0.0)
    u1 = jnp.maximum(dot(encb, w1e_ref[...]) + dot(agg1, w1a_ref[...]) + bu1_ref[...], 0.0)
    out_ref[...] = coef_ref[0] * u0 + coef_ref[1] * u1


def _update(enc, part, w0e, w0a, bu0, w1e, w1a, bu1, coef):
    bn = 1000
    grid = N // bn
    wspec = pl.BlockSpec((D, LD), lambda i: (0, 0))
    bspec = pl.BlockSpec((1, LD), lambda i: (0, 0))
    return pl.pallas_call(
        _update_body,
        grid=(grid,),
        in_specs=[
            pl.BlockSpec((bn, D), lambda i: (i, 0)),
            pl.BlockSpec((NC, 2, bn, LD), lambda i: (0, 0, i, 0)),
            wspec, wspec, bspec, wspec, wspec, bspec,
            pl.BlockSpec(memory_space=pltpu.SMEM),
        ],
        out_specs=pl.BlockSpec((bn, LD), lambda i: (i, 0)),
        out_shape=jax.ShapeDtypeStruct((N, LD), jnp.float32),
    )(enc, part, w0e, w0a, bu0, w1e, w1a, bu1, coef)


def kernel(z, e_feat, adj, enc,
           W_msg_0, b_msg_0, W_upd_0, b_upd_0,
           W_msg_1, b_msg_1, W_upd_1, b_upd_1,
           coef):
    src = jnp.pad(adj[0], (0, EPAD - E))
    dst = jnp.pad(adj[1], (0, EPAD - E))
    ef_pad = jnp.pad(e_feat, ((0, EPAD - E), (0, 0)))

    a0, b0, a1, b1 = _node_tables(z, W_msg_0[:D], W_msg_0[D:2 * D],
                                  W_msg_1[:D], W_msg_1[D:2 * D])
    c0, c1 = _edge_tables(ef_pad, W_msg_0[2 * D:], b_msg_0.reshape(1, LD),
                          W_msg_1[2 * D:], b_msg_1.reshape(1, LD))
    part = _edge_pass(src, dst, a0, b0, c0, a1, b1, c1)
    return _update(enc, part, W_upd_0[:D], W_upd_0[D:], b_upd_0.reshape(1, LD),
                   W_upd_1[:D], W_upd_1[D:], b_upd_1.reshape(1, LD), coef)


# speedup vs baseline: 2.0456x; 1.6170x over previous
import jax
import jax.numpy as jnp
from jax import lax
from jax.experimental import pallas as pl
from jax.experimental.pallas import tpu as pltpu
from jax.experimental.pallas import tpu_sc as plsc

N = 10000
E = 320000
D = 128
ED = 16
LD = 128

NC = 2
NS = 16
EPAD = 331776
K = 128
ET = EPAD // NS
NCHUNK = ET // K
NBUF = 3
NOUT = NCHUNK // NBUF
NPAD = 10112
NROW = NPAD // NS
EB = 2048

_NEG = -1e30


def _node_tables_body(z_ref, w0s_ref, w0d_ref, w1s_ref, w1d_ref,
                      a0_ref, b0_ref, a1_ref, b1_ref):
    zb = z_ref[...]
    dot = lambda w: lax.dot_general(zb, w, (((1,), (0,)), ((), ())),
                                    precision=lax.Precision.HIGHEST,
                                    preferred_element_type=jnp.float32)
    a0_ref[...] = dot(w0s_ref[...])
    b0_ref[...] = dot(w0d_ref[...])
    a1_ref[...] = dot(w1s_ref[...])
    b1_ref[...] = dot(w1d_ref[...])


def _node_tables(z, w0s, w0d, w1s, w1d):
    bn = 1000
    grid = N // bn
    wspec = pl.BlockSpec((D, LD), lambda i: (0, 0))
    ospec = pl.BlockSpec((bn, LD), lambda i: (i, 0))
    return pl.pallas_call(
        _node_tables_body,
        grid=(grid,),
        in_specs=[pl.BlockSpec((bn, D), lambda i: (i, 0)), wspec, wspec, wspec, wspec],
        out_specs=[ospec, ospec, ospec, ospec],
        out_shape=[jax.ShapeDtypeStruct((N, LD), jnp.float32)] * 4,
    )(z, w0s, w0d, w1s, w1d)


def _edge_tables_body(ef_ref, w0e_ref, b0_ref, w1e_ref, b1_ref, c0_ref, c1_ref):
    i = pl.program_id(0)
    eb = ef_ref[...]
    rows = i * EB + lax.broadcasted_iota(jnp.int32, (EB, 1), 0)
    valid = rows < E
    dot = lambda w: lax.dot_general(eb, w, (((1,), (0,)), ((), ())),
                                    precision=lax.Precision.HIGHEST,
                                    preferred_element_type=jnp.float32)
    c0 = dot(w0e_ref[...]) + b0_ref[...]
    c1 = dot(w1e_ref[...]) + b1_ref[...]
    c0_ref[...] = jnp.where(valid, c0, _NEG)
    c1_ref[...] = jnp.where(valid, c1, _NEG)


def _edge_tables(e_feat_pad, w0e, bm0, w1e, bm1):
    grid = EPAD // EB
    wspec = pl.BlockSpec((ED, LD), lambda i: (0, 0))
    bspec = pl.BlockSpec((1, LD), lambda i: (0, 0))
    ospec = pl.BlockSpec((EB, LD), lambda i: (i, 0))
    return pl.pallas_call(
        _edge_tables_body,
        grid=(grid,),
        in_specs=[pl.BlockSpec((EB, ED), lambda i: (i, 0)), wspec, bspec, wspec, bspec],
        out_specs=[ospec, ospec],
        out_shape=[jax.ShapeDtypeStruct((EPAD, LD), jnp.float32)] * 2,
    )(e_feat_pad, w0e, bm0, w1e, bm1)


def _edge_pass_body(adj_hbm, a0, b0, c0, a1, b1, c1, out_hbm,
                    idxb, msg, agg,
                    sem_i0, sem_i1, sem_i2, sem_c0, sem_c1, sem_c2,
                    sem_a0, sem_a1, sem_a2, sem_b0, sem_b1, sem_b2,
                    sem_s0, sem_s1, sem_s2):
    c = lax.axis_index("c")
    s = lax.axis_index("s")
    base = s * ET
    sem_i = (sem_i0, sem_i1, sem_i2)
    sem_c = (sem_c0, sem_c1, sem_c2)
    sem_a = (sem_a0, sem_a1, sem_a2)
    sem_b = (sem_b0, sem_b1, sem_b2)
    sem_s = (sem_s0, sem_s1, sem_s2)

    z16 = jnp.zeros((16,), jnp.float32)

    @pl.loop(0, K)
    def _z(r):
        for j in range(LD // 16):
            msg[0, r, pl.ds(j * 16, 16)] = z16

    for r in range(4):
        pltpu.sync_copy(msg.at[0], agg.at[pl.ds(s * NROW + r * K, K)])
    pltpu.sync_copy(msg.at[0, :NROW - 4 * K], agg.at[pl.ds(s * NROW + 4 * K, NROW - 4 * K)])
    plsc.subcore_barrier()

    def issue_loads(jj, b):
        e0 = base + jj * K
        pltpu.async_copy(adj_hbm.at[:, pl.ds(e0, K)], idxb.at[b], sem_i[b])

        @pl.when(c == 0)
        def _():
            pltpu.async_copy(c0.at[pl.ds(e0, K)], msg.at[b], sem_c[b])

        @pl.when(c == 1)
        def _():
            pltpu.async_copy(c1.at[pl.ds(e0, K)], msg.at[b], sem_c[b])

    issue_loads(0, 0)
    issue_loads(1, 1)
    issue_loads(2, 2)

    @pl.loop(0, NOUT)
    def _outer(g):
        for b in range(NBUF):
            jj = g * NBUF + b
            br = (b + 2) % NBUF
            pltpu.make_async_copy(adj_hbm.at[:, pl.ds(base, K)], idxb.at[b], sem_i[b]).wait()
            pltpu.make_async_copy(c0.at[pl.ds(base, K)], msg.at[b], sem_c[b]).wait()

            @pl.when(c == 0)
            def _():
                pltpu.async_copy(a0.at[idxb.at[b, 0]], msg.at[b], sem_a[b], add=True)
                pltpu.async_copy(b0.at[idxb.at[b, 1]], msg.at[b], sem_b[b], add=True)

            @pl.when(c == 1)
            def _():
                pltpu.async_copy(a1.at[idxb.at[b, 0]], msg.at[b], sem_a[b], add=True)
                pltpu.async_copy(b1.at[idxb.at[b, 1]], msg.at[b], sem_b[b], add=True)

            def _reload():
                pltpu.make_async_copy(msg.at[br], agg.at[idxb.at[br, 1]], sem_s[br]).wait()
                issue_loads(jj + 2, br)

            if b == 0:
                @pl.when(g >= 1)
                def _():
                    _reload()
            else:
                @pl.when(g < NOUT - 1)
                def _():
                    _reload()

            pltpu.make_async_copy(a0.at[idxb.at[b, 0]], msg.at[b], sem_a[b]).wait()
            pltpu.make_async_copy(b0.at[idxb.at[b, 1]], msg.at[b], sem_b[b]).wait()

            @pl.loop(0, K)
            def _relu(r):
                for j2 in range(LD // 16):
                    v = msg[b, r, pl.ds(j2 * 16, 16)]
                    msg[b, r, pl.ds(j2 * 16, 16)] = jnp.maximum(v, 0.0)

            pltpu.async_copy(msg.at[b], agg.at[idxb.at[b, 1]], sem_s[b], add=True)

    for b in range(NBUF):
        pltpu.make_async_copy(msg.at[b], agg.at[idxb.at[b, 1]], sem_s[b]).wait()

    plsc.subcore_barrier()
    pltpu.sync_copy(agg.at[pl.ds(s * NROW, NROW)],
                    out_hbm.at[c, pl.ds(s * NROW, NROW)])


def _edge_pass(adj_pad, a0, b0, c0, a1, b1, c1):
    mesh = plsc.VectorSubcoreMesh(core_axis_name="c", subcore_axis_name="s",
                                  num_cores=NC, num_subcores=NS)
    kern = pl.kernel(
        _edge_pass_body,
        out_type=jax.ShapeDtypeStruct((NC, NPAD, LD), jnp.float32),
        mesh=mesh,
        scratch_types=[
            pltpu.VMEM((NBUF, 2, K), jnp.int32),
            pltpu.VMEM((NBUF, K, LD), jnp.float32),
            pltpu.VMEM_SHARED((NPAD, LD), jnp.float32),
        ] + [pltpu.SemaphoreType.DMA] * 15,
    )
    return kern(adj_pad, a0, b0, c0, a1, b1, c1)


def _update_body(enc_ref, part_ref, w0e_ref, w0a_ref, bu0_ref,
                 w1e_ref, w1a_ref, bu1_ref, coef_ref, out_ref):
    encb = enc_ref[...]
    agg0 = part_ref[0]
    agg1 = part_ref[1]
    dot = lambda x, w: lax.dot_general(x, w, (((1,), (0,)), ((), ())),
                                       precision=lax.Precision.HIGHEST,
                                       preferred_element_type=jnp.float32)
    u0 = jnp.maximum(dot(encb, w0e_ref[...]) + dot(agg0, w0a_ref[...]) + bu0_ref[...], 0.0)
    u1 = jnp.maximum(dot(encb, w1e_ref[...]) + dot(agg1, w1a_ref[...]) + bu1_ref[...], 0.0)
    out_ref[...] = coef_ref[0] * u0 + coef_ref[1] * u1


def _update(enc, part, w0e, w0a, bu0, w1e, w1a, bu1, coef):
    bn = 1000
    grid = N // bn
    wspec = pl.BlockSpec((D, LD), lambda i: (0, 0))
    bspec = pl.BlockSpec((1, LD), lambda i: (0, 0))
    return pl.pallas_call(
        _update_body,
        grid=(grid,),
        in_specs=[
            pl.BlockSpec((bn, D), lambda i: (i, 0)),
            pl.BlockSpec((NC, bn, LD), lambda i: (0, i, 0)),
            wspec, wspec, bspec, wspec, wspec, bspec,
            pl.BlockSpec(memory_space=pltpu.SMEM),
        ],
        out_specs=pl.BlockSpec((bn, LD), lambda i: (i, 0)),
        out_shape=jax.ShapeDtypeStruct((N, LD), jnp.float32),
    )(enc, part, w0e, w0a, bu0, w1e, w1a, bu1, coef)


def kernel(z, e_feat, adj, enc,
           W_msg_0, b_msg_0, W_upd_0, b_upd_0,
           W_msg_1, b_msg_1, W_upd_1, b_upd_1,
           coef):
    adj_pad = jnp.pad(adj, ((0, 0), (0, EPAD - E)))
    ef_pad = jnp.pad(e_feat, ((0, EPAD - E), (0, 0)))

    a0, b0, a1, b1 = _node_tables(z, W_msg_0[:D], W_msg_0[D:2 * D],
                                  W_msg_1[:D], W_msg_1[D:2 * D])
    c0, c1 = _edge_tables(ef_pad, W_msg_0[2 * D:], b_msg_0.reshape(1, LD),
                          W_msg_1[2 * D:], b_msg_1.reshape(1, LD))
    part = _edge_pass(adj_pad, a0, b0, c0, a1, b1, c1)
    return _update(enc, part, W_upd_0[:D], W_upd_0[D:], b_upd_0.reshape(1, LD),
                   W_upd_1[:D], W_upd_1[D:], b_upd_1.reshape(1, LD), coef)


# speedup vs baseline: 2.3052x; 1.1269x over previous
import jax
import jax.numpy as jnp
from jax import lax
from jax.experimental import pallas as pl
from jax.experimental.pallas import tpu as pltpu
from jax.experimental.pallas import tpu_sc as plsc

N = 10000
E = 320000
D = 128
ED = 16
LD = 128

NC = 2
NS = 16
EPAD = 331776
K = 128
ET = EPAD // NS
NCHUNK = ET // K
NBUF = 3
NOUT = NCHUNK // NBUF
NPAD = 10112
NROW = NPAD // NS
EB = 2048

_NEG = -1e30


def _node_tables_body(z_ref, w0s_ref, w0d_ref, w1s_ref, w1d_ref,
                      a0_ref, b0_ref, a1_ref, b1_ref):
    zb = z_ref[...]
    dot = lambda w: lax.dot_general(zb, w, (((1,), (0,)), ((), ())),
                                    precision=lax.Precision.HIGHEST,
                                    preferred_element_type=jnp.float32)
    a0_ref[...] = dot(w0s_ref[...])
    b0_ref[...] = dot(w0d_ref[...])
    a1_ref[...] = dot(w1s_ref[...])
    b1_ref[...] = dot(w1d_ref[...])


def _node_tables(z, w0s, w0d, w1s, w1d):
    bn = 1000
    grid = N // bn
    wspec = pl.BlockSpec((D, LD), lambda i: (0, 0))
    ospec = pl.BlockSpec((bn, LD), lambda i: (i, 0))
    return pl.pallas_call(
        _node_tables_body,
        grid=(grid,),
        in_specs=[pl.BlockSpec((bn, D), lambda i: (i, 0)), wspec, wspec, wspec, wspec],
        out_specs=[ospec, ospec, ospec, ospec],
        out_shape=[jax.ShapeDtypeStruct((N, LD), jnp.float32)] * 4,
    )(z, w0s, w0d, w1s, w1d)


def _edge_tables_body(ef_ref, w0e_ref, b0_ref, w1e_ref, b1_ref, c0_ref, c1_ref):
    i = pl.program_id(0)
    eb = ef_ref[...]
    rows = i * EB + lax.broadcasted_iota(jnp.int32, (EB, 1), 0)
    valid = rows < E
    dot = lambda w: lax.dot_general(eb, w, (((1,), (0,)), ((), ())),
                                    precision=lax.Precision.HIGHEST,
                                    preferred_element_type=jnp.float32)
    c0 = dot(w0e_ref[...]) + b0_ref[...]
    c1 = dot(w1e_ref[...]) + b1_ref[...]
    c0_ref[...] = jnp.where(valid, c0, _NEG)
    c1_ref[...] = jnp.where(valid, c1, _NEG)


def _edge_tables(e_feat_pad, w0e, bm0, w1e, bm1):
    grid = EPAD // EB
    wspec = pl.BlockSpec((ED, LD), lambda i: (0, 0))
    bspec = pl.BlockSpec((1, LD), lambda i: (0, 0))
    ospec = pl.BlockSpec((EB, LD), lambda i: (i, 0))
    return pl.pallas_call(
        _edge_tables_body,
        grid=(grid,),
        in_specs=[pl.BlockSpec((EB, ED), lambda i: (i, 0)), wspec, bspec, wspec, bspec],
        out_specs=[ospec, ospec],
        out_shape=[jax.ShapeDtypeStruct((EPAD, LD), jnp.float32)] * 2,
    )(e_feat_pad, w0e, bm0, w1e, bm1)


def _edge_pass_body(adj_hbm, a0, b0, c0, a1, b1, c1, out_hbm,
                    idxb, msg, agg,
                    sem_i0, sem_i1, sem_i2, sem_c0, sem_c1, sem_c2,
                    sem_a0, sem_a1, sem_a2, sem_b0, sem_b1, sem_b2,
                    sem_s0, sem_s1, sem_s2):
    c = lax.axis_index("c")
    s = lax.axis_index("s")
    base = s * ET
    sem_i = (sem_i0, sem_i1, sem_i2)
    sem_c = (sem_c0, sem_c1, sem_c2)
    sem_a = (sem_a0, sem_a1, sem_a2)
    sem_b = (sem_b0, sem_b1, sem_b2)
    sem_s = (sem_s0, sem_s1, sem_s2)

    z16 = jnp.zeros((16,), jnp.float32)

    @pl.loop(0, K)
    def _z(r):
        for j in range(LD // 16):
            msg[0, r, pl.ds(j * 16, 16)] = z16

    for r in range(4):
        pltpu.sync_copy(msg.at[0], agg.at[pl.ds(s * NROW + r * K, K)])
    pltpu.sync_copy(msg.at[0, :NROW - 4 * K], agg.at[pl.ds(s * NROW + 4 * K, NROW - 4 * K)])
    plsc.subcore_barrier()

    def issue_loads(jj, b):
        e0 = base + jj * K
        pltpu.async_copy(adj_hbm.at[:, pl.ds(e0, K)], idxb.at[b], sem_i[b])

        @pl.when(c == 0)
        def _():
            pltpu.async_copy(c0.at[pl.ds(e0, K)], msg.at[b], sem_c[b])

        @pl.when(c == 1)
        def _():
            pltpu.async_copy(c1.at[pl.ds(e0, K)], msg.at[b], sem_c[b])

    def wait_loads_issue_gathers(q, sq):
        pltpu.make_async_copy(adj_hbm.at[:, pl.ds(base, K)], idxb.at[sq], sem_i[sq]).wait()
        pltpu.make_async_copy(c0.at[pl.ds(base, K)], msg.at[sq], sem_c[sq]).wait()

        @pl.when(c == 0)
        def _():
            pltpu.async_copy(a0.at[idxb.at[sq, 0]], msg.at[sq], sem_a[sq], add=True)
            pltpu.async_copy(b0.at[idxb.at[sq, 1]], msg.at[sq], sem_b[sq], add=True)

        @pl.when(c == 1)
        def _():
            pltpu.async_copy(a1.at[idxb.at[sq, 0]], msg.at[sq], sem_a[sq], add=True)
            pltpu.async_copy(b1.at[idxb.at[sq, 1]], msg.at[sq], sem_b[sq], add=True)

    issue_loads(0, 0)
    issue_loads(1, 1)
    issue_loads(2, 2)
    wait_loads_issue_gathers(0, 0)

    @pl.loop(0, NOUT)
    def _outer(g):
        for b in range(NBUF):
            jj = g * NBUF + b
            b1 = (b + 1) % NBUF
            br = (b + 2) % NBUF

            if b == NBUF - 1:
                @pl.when(g < NOUT - 1)
                def _():
                    wait_loads_issue_gathers(jj + 1, b1)
            else:
                wait_loads_issue_gathers(jj + 1, b1)

            def _reload():
                pltpu.make_async_copy(msg.at[br], agg.at[idxb.at[br, 1]], sem_s[br]).wait()
                issue_loads(jj + 2, br)

            if b == 0:
                @pl.when(g >= 1)
                def _():
                    _reload()
            else:
                @pl.when(g < NOUT - 1)
                def _():
                    _reload()

            pltpu.make_async_copy(a0.at[idxb.at[b, 0]], msg.at[b], sem_a[b]).wait()
            pltpu.make_async_copy(b0.at[idxb.at[b, 1]], msg.at[b], sem_b[b]).wait()

            @pl.loop(0, K)
            def _relu(r):
                for j2 in range(LD // 16):
                    v = msg[b, r, pl.ds(j2 * 16, 16)]
                    msg[b, r, pl.ds(j2 * 16, 16)] = jnp.maximum(v, 0.0)

            pltpu.async_copy(msg.at[b], agg.at[idxb.at[b, 1]], sem_s[b], add=True)

    for b in range(NBUF):
        pltpu.make_async_copy(msg.at[b], agg.at[idxb.at[b, 1]], sem_s[b]).wait()

    plsc.subcore_barrier()
    pltpu.sync_copy(agg.at[pl.ds(s * NROW, NROW)],
                    out_hbm.at[c, pl.ds(s * NROW, NROW)])


def _edge_pass(adj_pad, a0, b0, c0, a1, b1, c1):
    mesh = plsc.VectorSubcoreMesh(core_axis_name="c", subcore_axis_name="s",
                                  num_cores=NC, num_subcores=NS)
    kern = pl.kernel(
        _edge_pass_body,
        out_type=jax.ShapeDtypeStruct((NC, NPAD, LD), jnp.float32),
        mesh=mesh,
        scratch_types=[
            pltpu.VMEM((NBUF, 2, K), jnp.int32),
            pltpu.VMEM((NBUF, K, LD), jnp.float32),
            pltpu.VMEM_SHARED((NPAD, LD), jnp.float32),
        ] + [pltpu.SemaphoreType.DMA] * 15,
    )
    return kern(adj_pad, a0, b0, c0, a1, b1, c1)


def _update_body(enc_ref, part_ref, w0e_ref, w0a_ref, bu0_ref,
                 w1e_ref, w1a_ref, bu1_ref, coef_ref, out_ref):
    encb = enc_ref[...]
    agg0 = part_ref[0]
    agg1 = part_ref[1]
    dot = lambda x, w: lax.dot_general(x, w, (((1,), (0,)), ((), ())),
                                       precision=lax.Precision.HIGHEST,
                                       preferred_element_type=jnp.float32)
    u0 = jnp.maximum(dot(encb, w0e_ref[...]) + dot(agg0, w0a_ref[...]) + bu0_ref[...], 0.0)
    u1 = jnp.maximum(dot(encb, w1e_ref[...]) + dot(agg1, w1a_ref[...]) + bu1_ref[...], 0.0)
    out_ref[...] = coef_ref[0] * u0 + coef_ref[1] * u1


def _update(enc, part, w0e, w0a, bu0, w1e, w1a, bu1, coef):
    bn = 1000
    grid = N // bn
    wspec = pl.BlockSpec((D, LD), lambda i: (0, 0))
    bspec = pl.BlockSpec((1, LD), lambda i: (0, 0))
    return pl.pallas_call(
        _update_body,
        grid=(grid,),
        in_specs=[
            pl.BlockSpec((bn, D), lambda i: (i, 0)),
            pl.BlockSpec((NC, bn, LD), lambda i: (0, i, 0)),
            wspec, wspec, bspec, wspec, wspec, bspec,
            pl.BlockSpec(memory_space=pltpu.SMEM),
        ],
        out_specs=pl.BlockSpec((bn, LD), lambda i: (i, 0)),
        out_shape=jax.ShapeDtypeStruct((N, LD), jnp.float32),
    )(enc, part, w0e, w0a, bu0, w1e, w1a, bu1, coef)


def kernel(z, e_feat, adj, enc,
           W_msg_0, b_msg_0, W_upd_0, b_upd_0,
           W_msg_1, b_msg_1, W_upd_1, b_upd_1,
           coef):
    adj_pad = jnp.pad(adj, ((0, 0), (0, EPAD - E)))
    ef_pad = jnp.pad(e_feat, ((0, EPAD - E), (0, 0)))

    a0, b0, a1, b1 = _node_tables(z, W_msg_0[:D], W_msg_0[D:2 * D],
                                  W_msg_1[:D], W_msg_1[D:2 * D])
    c0, c1 = _edge_tables(ef_pad, W_msg_0[2 * D:], b_msg_0.reshape(1, LD),
                          W_msg_1[2 * D:], b_msg_1.reshape(1, LD))
    part = _edge_pass(adj_pad, a0, b0, c0, a1, b1, c1)
    return _update(enc, part, W_upd_0[:D], W_upd_0[D:], b_upd_0.reshape(1, LD),
                   W_upd_1[:D], W_upd_1[D:], b_upd_1.reshape(1, LD), coef)
